# baseline (speedup 1.0000x reference)
import functools

import jax
import jax.numpy as jnp
from jax.experimental import pallas as pl
from jax.experimental.pallas import tpu as pltpu

_HID = 64
_CUTOFF = 2.5
_G = 8
_TC = 8

_PER_LAYER = 18


def _flatten_layer(lp):
    H = _HID
    return [
        lp["e_w1"][:, :H].T,
        lp["e_w1"][:, H:2 * H].T,
        lp["e_w1"][:, 2 * H].reshape(1, -1),
        lp["e_b1"].reshape(1, -1),
        lp["e_w2"].T,
        lp["e_b2"].reshape(1, -1),
        lp["g_w"].reshape(1, -1),
        lp["g_b"].reshape(1, 1),
        lp["c_w1"].T,
        lp["c_b1"].reshape(1, -1),
        lp["c_w2"].reshape(1, -1),
        lp["c_b2"].reshape(1, 1),
        lp["n_w1"][:, :H].T,
        lp["n_w1"][:, H:].T,
        lp["n_b1"].reshape(1, -1),
        lp["n_w2"].T,
        lp["n_b2"].reshape(1, -1),
        lp["coors_scale"].reshape(1, 1),
    ]


def _mm(a, b):
    return jax.lax.dot_general(
        a.astype(jnp.bfloat16), b.astype(jnp.bfloat16),
        (((a.ndim - 1,), (0,)), ((), ())),
        preferred_element_type=jnp.float32)


def _mmf(a, b):
    return jax.lax.dot_general(
        a, b, (((a.ndim - 1,), (a.ndim - 2,)), ((), ())),
        preferred_element_type=jnp.float32)


def _bmm(a, b):
    return jax.lax.dot_general(
        a, b, (((2,), (1,)), ((0,), (0,))),
        preferred_element_type=jnp.float32)


def _fused_kernel(n_layers, kmax_ref, *refs):
    atom_ref, pos_ref, mask_ref, mask_col_ref = refs[0], refs[1], refs[2], refs[3]
    emb_w_ref, emb_b_ref, ho_w_ref, ho_b_ref, am_w_ref, am_b_ref = refs[4:10]
    layer_refs = refs[10:10 + n_layers * _PER_LAYER]
    h_out_ref, x_out_ref = refs[-6], refs[-5]
    mi_acc_ref, x0a_ref, x1a_ref, x2a_ref = refs[-4:]

    G, n, in_nf = atom_ref.shape
    H = _HID
    T = _TC
    NT = n // T
    kmax = kmax_ref[pl.program_id(0)]

    atom = atom_ref[...]
    pos = pos_ref[...]
    mask = mask_ref[...]

    a2 = atom.reshape(G * n, in_nf)
    h = _mm(a2, emb_w_ref[...]) + emb_b_ref[...]

    px = pos[:, :, 0]
    py = pos[:, :, 1]
    pz = pos[:, :, 2]

    r0 = px[:, :, None] - px[:, None, :]
    r1 = py[:, :, None] - py[:, None, :]
    r2 = pz[:, :, None] - pz[:, None, :]
    d0 = jnp.sqrt(r0 * r0 + r1 * r1 + r2 * r2)
    ii = jax.lax.broadcasted_iota(jnp.int32, (n, n), 0)
    jj = jax.lax.broadcasted_iota(jnp.int32, (n, n), 1)
    not_self = (ii != jj).astype(jnp.float32)[None]
    em_full = jnp.where(d0 < _CUTOFF, 1.0, 0.0) * not_self \
        * mask[:, :, None] * mask[:, None, :]

    lt = (ii > jj).astype(jnp.float32)
    rank_row = jax.lax.dot_general(
        mask, lt, (((1,), (0,)), ((), ())),
        preferred_element_type=jnp.float32)
    k_lane = jax.lax.broadcasted_iota(jnp.int32, (1, 1, n), 2).astype(jnp.float32)
    k_sub = jax.lax.broadcasted_iota(jnp.int32, (1, n, 1), 1).astype(jnp.float32)
    Pk = mask[:, None, :] * (rank_row[:, None, :] == k_sub).astype(jnp.float32)
    rank_col = rank_row.reshape(G, n, 1)
    PkT = mask.reshape(G, n, 1) * (rank_col == k_lane).astype(jnp.float32)

    emc = _bmm(_bmm(Pk, em_full), PkT)
    eyeT = jnp.eye(T, dtype=jnp.float32).reshape(1, 1, T, T)

    def to_col4(x, s):
        return jnp.sum(x[:, :, None, :] * s, axis=-1, keepdims=True)

    def to_pair(x4, s):
        return jnp.sum(x4 * s, axis=2)

    x0 = _bmm(Pk, pos[:, :, 0:1])[:, :, 0]
    x1 = _bmm(Pk, pos[:, :, 1:2])[:, :, 0]
    x2 = _bmm(Pk, pos[:, :, 2:3])[:, :, 0]

    for li in range(n_layers):
        (Wa, Wb, wd, b1, W2, b2, gv, gb, C1, cb1, c2, cb2,
         Wh, Wm, nb1, N2, nb2, cs) = (
            r[...] for r in layer_refs[li * _PER_LAYER:(li + 1) * _PER_LAYER])

        h3 = h.reshape(G, n, H)
        hc = _bmm(Pk, h3)
        hcf = hc.reshape(G * n, H)
        hA = _mm(hcf, Wa).reshape(G, n, -1)
        hB = _mm(hcf, Wb).reshape(G, n, -1)

        mi_acc_ref[...] = jnp.zeros((G, n, H), jnp.float32)
        x0a_ref[...] = jnp.zeros((G, n), jnp.float32)
        x1a_ref[...] = jnp.zeros((G, n), jnp.float32)
        x2a_ref[...] = jnp.zeros((G, n), jnp.float32)

        for ci in range(NT):
            for cj in range(NT):
                @pl.when((ci * T < kmax) & (cj * T < kmax))
                def _tile(ci=ci, cj=cj):
                    si, sj = ci * T, cj * T
                    hA_t = hA[:, si:si + T, None, :]
                    hB_t = hB[:, None, sj:sj + T, :]
                    x0i, x0j = x0[:, si:si + T], x0[:, sj:sj + T]
                    x1i, x1j = x1[:, si:si + T], x1[:, sj:sj + T]
                    x2i, x2j = x2[:, si:si + T], x2[:, sj:sj + T]
                    r0t = x0i[:, :, None] - x0j[:, None, :]
                    r1t = x1i[:, :, None] - x1j[:, None, :]
                    r2t = x2i[:, :, None] - x2j[:, None, :]
                    distt = r0t * r0t + r1t * r1t + r2t * r2t
                    dist4t = to_col4(distt, eyeT)
                    emt = emc[:, si:si + T, sj:sj + T]
                    em4t = to_col4(emt, eyeT)

                    m1 = (hA_t + hB_t + dist4t * wd.reshape(1, 1, 1, -1)
                          + b1.reshape(1, 1, 1, -1))
                    m1 = jax.nn.silu(m1).reshape(G * T * T, -1)
                    m = jax.nn.silu(_mm(m1, W2) + b2)
                    t = jax.nn.silu(_mm(m, C1) + cb1)

                    t4 = t.reshape(G, T, T, -1)
                    cw4 = jnp.sum(t4 * c2.reshape(1, 1, 1, -1), axis=-1,
                                  keepdims=True) + cb2.reshape(1, 1, 1, 1)
                    m4 = m.reshape(G, T, T, H)
                    gate4 = jax.nn.sigmoid(
                        jnp.sum(m4 * gv.reshape(1, 1, 1, -1), axis=-1,
                                keepdims=True) + gb.reshape(1, 1, 1, 1))
                    mg4 = m4 * (gate4 * em4t)
                    mi_acc_ref[:, si:si + T, :] += jnp.sum(mg4, axis=2)

                    cwt = to_pair(cw4, eyeT)
                    nrmt = jnp.sqrt(distt)
                    invt = cs.reshape(1, 1, 1) / jnp.clip(nrmt, 1e-8, None)
                    wgt = cwt * emt * invt
                    x0a_ref[:, si:si + T] += jnp.sum(wgt * r0t, axis=2)
                    x1a_ref[:, si:si + T] += jnp.sum(wgt * r1t, axis=2)
                    x2a_ref[:, si:si + T] += jnp.sum(wgt * r2t, axis=2)

        x0 = x0 + x0a_ref[...]
        x1 = x1 + x1a_ref[...]
        x2 = x2 + x2a_ref[...]

        m_i = _bmm(PkT, mi_acc_ref[...]).reshape(G * n, H)
        nh = jax.nn.silu(_mm(h, Wh) + _mm(m_i, Wm) + nb1)
        h = h + _mm(nh, N2) + nb2

    h_out = _mm(h, ho_w_ref[...]) + ho_b_ref[...]
    atom_all = _mm(a2, am_w_ref[...]) + am_b_ref[...]
    mflat = mask_col_ref[...]
    h_full = jnp.where(mflat > 0.0, h_out, atom_all).reshape(G, n, H)

    xf0 = _bmm(PkT, x0[:, :, None])[:, :, 0] + (1.0 - mask) * px
    xf1 = _bmm(PkT, x1[:, :, None])[:, :, 0] + (1.0 - mask) * py
    xf2 = _bmm(PkT, x2[:, :, None])[:, :, 0] + (1.0 - mask) * pz
    x_full = jnp.concatenate(
        [xf0[:, :, None], xf1[:, :, None], xf2[:, :, None]], axis=-1)

    h_out_ref[...] = h_full
    x_out_ref[...] = x_full


@jax.jit
def kernel(ligand_atom, ligand_pos, ligand_pad_mask, params):
    bs, n, in_nf = ligand_atom.shape
    H = _HID
    p = params
    n_layers = len(p["layers"])
    G = _G
    nblk = bs // G

    counts = ligand_pad_mask.sum(axis=1).astype(jnp.int32)
    order = jnp.argsort(counts)
    inv_order = jnp.argsort(order)
    atom_s = ligand_atom[order]
    pos_s = ligand_pos[order]
    mask_s = ligand_pad_mask[order]
    kmax_blk = jnp.max(counts[order].reshape(nblk, G), axis=1)

    weights = [
        p["atom_emb_w"].T, p["atom_emb_b"].reshape(1, -1),
        p["h_out_w"].T, p["h_out_b"].reshape(1, -1),
        p["atom_mlp_w"].T, p["atom_mlp_b"].reshape(1, -1),
    ]
    for lp in p["layers"]:
        weights += _flatten_layer(lp)
    weights = [w.astype(jnp.float32) for w in weights]

    mask_f = mask_s.astype(jnp.float32)
    mask_col = mask_f.reshape(bs * n, 1)

    def batch_spec(shape):
        blk = (G,) + shape
        return pl.BlockSpec(blk, lambda b, kr: (b,) + (0,) * len(shape))

    def full_spec(w):
        nd = w.ndim
        return pl.BlockSpec(w.shape, lambda b, kr, _nd=nd: (0,) * _nd)

    in_specs = [
        batch_spec((n, in_nf)),
        batch_spec((n, 3)),
        batch_spec((n,)),
        pl.BlockSpec((G * n, 1), lambda b, kr: (b, 0)),
    ] + [full_spec(w) for w in weights]

    out_specs = [batch_spec((n, H)), batch_spec((n, 3))]
    out_shapes = [
        jax.ShapeDtypeStruct((bs, n, H), jnp.float32),
        jax.ShapeDtypeStruct((bs, n, 3), jnp.float32),
    ]
    scratch = [
        pltpu.VMEM((G, n, H), jnp.float32),
        pltpu.VMEM((G, n), jnp.float32),
        pltpu.VMEM((G, n), jnp.float32),
        pltpu.VMEM((G, n), jnp.float32),
    ]

    h_full, x_full = pl.pallas_call(
        functools.partial(_fused_kernel, n_layers),
        grid_spec=pltpu.PrefetchScalarGridSpec(
            num_scalar_prefetch=1,
            grid=(nblk,),
            in_specs=in_specs,
            out_specs=out_specs,
            scratch_shapes=scratch,
        ),
        out_shape=out_shapes,
    )(kmax_blk, atom_s, pos_s, mask_f, mask_col, *weights)

    h_full = h_full[inv_order]
    x_full = x_full[inv_order]

    x_full = jnp.where(jnp.any(jnp.isnan(x_full)),
                       jnp.zeros_like(x_full), x_full)
    return h_full, x_full


# speedup vs baseline: 1.2911x; 1.2911x over previous
import functools

import jax
import jax.numpy as jnp
from jax.experimental import pallas as pl

_HID = 64
_CUTOFF = 2.5
_G = 8


def _mm(a, b):
    return jax.lax.dot_general(
        a.astype(jnp.bfloat16), b.astype(jnp.bfloat16),
        (((a.ndim - 1,), (0,)), ((), ())),
        preferred_element_type=jnp.float32)

_PER_LAYER = 18


def _flatten_layer(lp):
    H = _HID
    return [
        lp["e_w1"][:, :H].T,
        lp["e_w1"][:, H:2 * H].T,
        lp["e_w1"][:, 2 * H].reshape(1, -1),
        lp["e_b1"].reshape(1, -1),
        lp["e_w2"].T,
        lp["e_b2"].reshape(1, -1),
        lp["g_w"].reshape(1, -1),
        lp["g_b"].reshape(1, 1),
        lp["c_w1"].T,
        lp["c_b1"].reshape(1, -1),
        lp["c_w2"].reshape(1, -1),
        lp["c_b2"].reshape(1, 1),
        lp["n_w1"][:, :H].T,
        lp["n_w1"][:, H:].T,
        lp["n_b1"].reshape(1, -1),
        lp["n_w2"].T,
        lp["n_b2"].reshape(1, -1),
        lp["coors_scale"].reshape(1, 1),
    ]


def _fused_kernel(n_layers, *refs):
    atom_ref, pos_ref, mask_ref, mask_col_ref = refs[0], refs[1], refs[2], refs[3]
    emb_w_ref, emb_b_ref, ho_w_ref, ho_b_ref, am_w_ref, am_b_ref = refs[4:10]
    layer_refs = refs[10:10 + n_layers * _PER_LAYER]
    h_out_ref, x_out_ref = refs[-2], refs[-1]

    G, n, in_nf = atom_ref.shape
    H = _HID

    atom = atom_ref[...]
    pos = pos_ref[...]
    mask = mask_ref[...]

    a2 = atom.reshape(G * n, in_nf)
    h = _mm(a2, emb_w_ref[...]) + emb_b_ref[...]

    px = pos[:, :, 0]
    py = pos[:, :, 1]
    pz = pos[:, :, 2]

    r0 = px[:, :, None] - px[:, None, :]
    r1 = py[:, :, None] - py[:, None, :]
    r2 = pz[:, :, None] - pz[:, None, :]
    d0 = jnp.sqrt(r0 * r0 + r1 * r1 + r2 * r2)
    ii = jax.lax.broadcasted_iota(jnp.int32, (n, n), 0)
    jj = jax.lax.broadcasted_iota(jnp.int32, (n, n), 1)
    not_self = (ii != jj).astype(jnp.float32)[None]
    em = jnp.where(d0 < _CUTOFF, 1.0, 0.0) * not_self \
        * mask[:, :, None] * mask[:, None, :]
    eye4 = (ii == jj).astype(jnp.float32)[None, None]

    def to_col4(x):
        return jnp.sum(x[:, :, None, :] * eye4, axis=-1, keepdims=True)

    def to_pair(x4):
        return jnp.sum(x4 * eye4, axis=2)

    em4 = to_col4(em)

    x0, x1, x2 = px, py, pz
    for li in range(n_layers):
        (Wa, Wb, wd, b1, W2, b2, gv, gb, C1, cb1, c2, cb2,
         Wh, Wm, nb1, N2, nb2, cs) = (
            r[...] for r in layer_refs[li * _PER_LAYER:(li + 1) * _PER_LAYER])

        r0 = x0[:, :, None] - x0[:, None, :]
        r1 = x1[:, :, None] - x1[:, None, :]
        r2 = x2[:, :, None] - x2[:, None, :]
        dist = r0 * r0 + r1 * r1 + r2 * r2
        dist4 = to_col4(dist)

        hA = _mm(h, Wa)
        hB = _mm(h, Wb)
        m1 = (hA.reshape(G, n, 1, -1) + hB.reshape(G, 1, n, -1)
              + dist4 * wd.reshape(1, 1, 1, -1) + b1.reshape(1, 1, 1, -1))
        m1 = jax.nn.silu(m1).reshape(G * n * n, -1)
        m = jax.nn.silu(_mm(m1, W2) + b2)
        t = jax.nn.silu(_mm(m, C1) + cb1)

        t4 = t.reshape(G, n, n, -1)
        cw4 = jnp.sum(t4 * c2.reshape(1, 1, 1, -1), axis=-1, keepdims=True) \
            + cb2.reshape(1, 1, 1, 1)

        m4 = m.reshape(G, n, n, H)
        gate4 = jax.nn.sigmoid(
            jnp.sum(m4 * gv.reshape(1, 1, 1, -1), axis=-1, keepdims=True)
            + gb.reshape(1, 1, 1, 1))
        mg4 = m4 * (gate4 * em4)
        m_i = jnp.sum(mg4, axis=2).reshape(G * n, H)

        cw = to_pair(cw4)
        nrm = jnp.sqrt(dist)
        inv = cs.reshape(1, 1, 1) / jnp.clip(nrm, 1e-8, None)
        wgt = cw * em * inv
        x0 = x0 + jnp.sum(wgt * r0, axis=2)
        x1 = x1 + jnp.sum(wgt * r1, axis=2)
        x2 = x2 + jnp.sum(wgt * r2, axis=2)

        nh = jax.nn.silu(_mm(h, Wh) + _mm(m_i, Wm) + nb1)
        h = h + _mm(nh, N2) + nb2

    h_out = _mm(h, ho_w_ref[...]) + ho_b_ref[...]
    atom_all = _mm(a2, am_w_ref[...]) + am_b_ref[...]
    mflat = mask_col_ref[...]
    h_full = jnp.where(mflat > 0.0, h_out, atom_all).reshape(G, n, H)

    keep = mask > 0.0
    xf0 = jnp.where(keep, x0, px)
    xf1 = jnp.where(keep, x1, py)
    xf2 = jnp.where(keep, x2, pz)
    x_full = jnp.concatenate(
        [xf0[:, :, None], xf1[:, :, None], xf2[:, :, None]], axis=-1)

    h_out_ref[...] = h_full
    x_out_ref[...] = x_full


@jax.jit
def kernel(ligand_atom, ligand_pos, ligand_pad_mask, params):
    bs, n, in_nf = ligand_atom.shape
    H = _HID
    p = params
    n_layers = len(p["layers"])

    weights = [
        p["atom_emb_w"].T, p["atom_emb_b"].reshape(1, -1),
        p["h_out_w"].T, p["h_out_b"].reshape(1, -1),
        p["atom_mlp_w"].T, p["atom_mlp_b"].reshape(1, -1),
    ]
    for lp in p["layers"]:
        weights += _flatten_layer(lp)
    weights = [w.astype(jnp.float32) for w in weights]

    mask_f = ligand_pad_mask.astype(jnp.float32)
    mask_col = mask_f.reshape(bs * n, 1)

    G = _G
    grid = (bs // G,)

    def batch_spec(shape):
        blk = (G,) + shape
        return pl.BlockSpec(blk, lambda b: (b,) + (0,) * len(shape))

    def full_spec(w):
        nd = w.ndim
        return pl.BlockSpec(w.shape, lambda b, _nd=nd: (0,) * _nd)

    in_specs = [
        batch_spec((n, in_nf)),
        batch_spec((n, 3)),
        batch_spec((n,)),
        pl.BlockSpec((G * n, 1), lambda b: (b, 0)),
    ] + [full_spec(w) for w in weights]

    out_specs = [batch_spec((n, H)), batch_spec((n, 3))]
    out_shapes = [
        jax.ShapeDtypeStruct((bs, n, H), jnp.float32),
        jax.ShapeDtypeStruct((bs, n, 3), jnp.float32),
    ]

    h_full, x_full = pl.pallas_call(
        functools.partial(_fused_kernel, n_layers),
        grid=grid,
        in_specs=in_specs,
        out_specs=out_specs,
        out_shape=out_shapes,
    )(ligand_atom, ligand_pos, mask_f, mask_col, *weights)

    x_full = jnp.where(jnp.any(jnp.isnan(x_full)),
                       jnp.zeros_like(x_full), x_full)
    return h_full, x_full
